# baseline (speedup 1.0000x reference)
import functools

import jax
import jax.numpy as jnp
from jax import lax
from jax.experimental import pallas as pl
from jax.experimental.pallas import tpu as pltpu
from jax.experimental.pallas import tpu_sc as plsc

N = 10000
D = 128
E = 320000
NC = 2
NS = 16
NW = NC * NS
EPT = E // NW
CH = 64
NCHUNK = 160
Q = 3
PADE = NCHUNK * CH - EPT
N_PAD = 10240
RPT = N_PAD // NS
EPS = 1e-5


@functools.cache
def _get_sc_deg():
  mesh = plsc.VectorSubcoreMesh(core_axis_name="c", subcore_axis_name="s")

  @functools.partial(
      pl.kernel,
      out_type=jax.ShapeDtypeStruct((NC, N_PAD), jnp.float32),
      mesh=mesh,
      scratch_types=[
          pltpu.VMEM((NCHUNK, CH), jnp.int32),
          pltpu.VMEM((CH,), jnp.float32),
          pltpu.VMEM((RPT,), jnp.float32),
          pltpu.VMEM_SHARED((N_PAD,), jnp.float32),
      ],
  )
  def _sc_deg(dst_hbm, out_hbm, dst_v, ones_v, zbuf_v, acc_sh):
    c = lax.axis_index("c")
    s = lax.axis_index("s")
    wid = c * NS + s

    pltpu.sync_copy(dst_hbm.at[wid], dst_v)

    one16 = jnp.ones((16,), jnp.float32)
    z16 = jnp.zeros((16,), jnp.float32)
    for k in range(CH // 16):
      ones_v[pl.ds(k * 16, 16)] = one16

    def _zero(i, carry):
      zbuf_v[pl.ds(i * 16, 16)] = z16
      return carry

    lax.fori_loop(0, RPT // 16, _zero, 0)
    pltpu.sync_copy(zbuf_v, acc_sh.at[pl.ds(s * RPT, RPT)])
    plsc.subcore_barrier()

    def _chunk(j, carry):
      pltpu.sync_copy(ones_v, acc_sh.at[dst_v.at[j]], add=True)
      return carry

    lax.fori_loop(0, NCHUNK, _chunk, 0)
    plsc.subcore_barrier()

    pltpu.sync_copy(acc_sh.at[pl.ds(s * RPT, RPT)],
                    out_hbm.at[c, pl.ds(s * RPT, RPT)])

  return _sc_deg


@functools.cache
def _get_sc_agg():
  mesh = plsc.VectorSubcoreMesh(core_axis_name="c", subcore_axis_name="s")

  @functools.partial(
      pl.kernel,
      out_type=jax.ShapeDtypeStruct((NC, N_PAD, D), jnp.float32),
      mesh=mesh,
      scratch_types=(
          [pltpu.VMEM((CH,), jnp.int32) for _ in range(Q)]
          + [pltpu.VMEM((CH, D), jnp.float32) for _ in range(Q)]
          + [pltpu.VMEM((NCHUNK, CH), jnp.int32)]
          + [pltpu.VMEM_SHARED((N_PAD, D), jnp.float32)]
          + [pltpu.SemaphoreType.DMA] * (2 * Q)
      ),
  )
  def _sc_agg(hs_hbm, src_hbm, dst_hbm, out_hbm, *refs):
    sidx = refs[0:Q]
    rows = refs[Q:2 * Q]
    dst_v = refs[2 * Q]
    acc_sh = refs[2 * Q + 1]
    sg = refs[2 * Q + 2:2 * Q + 2 + Q]
    si = refs[2 * Q + 2 + Q:2 * Q + 2 + 2 * Q]

    c = lax.axis_index("c")
    s = lax.axis_index("s")
    wid = c * NS + s

    pltpu.sync_copy(dst_hbm.at[wid], dst_v)

    z16 = jnp.zeros((16,), jnp.float32)

    def _zero(i, carry):
      for k in range(D // 16):
        rows[0][i, pl.ds(k * 16, 16)] = z16
      return carry

    lax.fori_loop(0, CH, _zero, 0)
    for m in range(RPT // CH):
      pltpu.sync_copy(rows[0], acc_sh.at[pl.ds(s * RPT + m * CH, CH)])
    if RPT % CH:
      pltpu.sync_copy(
          rows[0].at[pl.ds(0, RPT % CH)],
          acc_sh.at[pl.ds(s * RPT + (RPT // CH) * CH, RPT % CH)])
    plsc.subcore_barrier()

    for q in range(Q - 1):
      pltpu.sync_copy(src_hbm.at[wid, q], sidx[q])
      pltpu.async_copy(hs_hbm.at[sidx[q]], rows[q], sg[q])
    pltpu.async_copy(src_hbm.at[wid, Q - 1], sidx[Q - 1], si[Q - 1])

    def _body(i, carry):
      for k in range(Q):
        t = i * Q + k
        kg = (k + Q - 1) % Q
        tg = jnp.minimum(t + Q - 1, NCHUNK - 1)
        tp = jnp.minimum(t + Q, NCHUNK - 1)
        pltpu.make_async_copy(src_hbm.at[wid, tg], sidx[kg], si[kg]).wait()
        pltpu.async_copy(hs_hbm.at[sidx[kg]], rows[kg], sg[kg])
        pltpu.make_async_copy(hs_hbm.at[sidx[k]], rows[k], sg[k]).wait()
        pltpu.async_copy(src_hbm.at[wid, tp], sidx[k], si[k])
        pltpu.sync_copy(rows[k], acc_sh.at[dst_v.at[t]], add=True)
      return carry

    lax.fori_loop(0, NCHUNK // Q, _body, 0)
    for q in range(Q - 1):
      pltpu.make_async_copy(hs_hbm.at[sidx[q]], rows[q], sg[q]).wait()
    pltpu.make_async_copy(src_hbm.at[wid, 0], sidx[Q - 1], si[Q - 1]).wait()
    plsc.subcore_barrier()

    pltpu.sync_copy(acc_sh.at[pl.ds(s * RPT, RPT)],
                    out_hbm.at[c, pl.ds(s * RPT, RPT)])

  return _sc_agg


def _tc_dinv(partials):
  def body(p_ref, o_ref):
    p = p_ref[...]
    deg = p[0:1, :] + p[1:2, :] + 1.0
    col = lax.broadcasted_iota(jnp.int32, (1, N_PAD), 1)
    o_ref[...] = jnp.where(col < N, lax.rsqrt(deg), 0.0)

  return pl.pallas_call(
      body, out_shape=jax.ShapeDtypeStruct((1, N_PAD), jnp.float32)
  )(partials)


def _tc_pre(x_pad, W, dcol):
  def body(x_ref, w_ref, d_ref, o_ref):
    h = jnp.dot(x_ref[...], w_ref[...], preferred_element_type=jnp.float32)
    o_ref[...] = h * d_ref[...]

  return pl.pallas_call(
      body, out_shape=jax.ShapeDtypeStruct((N_PAD, D), jnp.float32)
  )(x_pad, W, dcol)


def _tc_mid(p0, p1, hs, dcol, b, g, be, Wn):
  def body(p0r, p1r, hsr, dr, br, gr, ber, wr, o_ref):
    t = (p0r[...] + p1r[...] + hsr[...]) * dr[...] + br[...]
    row = lax.broadcasted_iota(jnp.int32, (N_PAD, 1), 0)
    m = jnp.where(row < N, 1.0, 0.0)
    mean = jnp.sum(t * m, axis=0, keepdims=True) * (1.0 / N)
    ctr = (t - mean) * m
    var = jnp.sum(ctr * ctr, axis=0, keepdims=True) * (1.0 / N)
    y = jnp.maximum(gr[...] * ctr * lax.rsqrt(var + EPS) + ber[...], 0.0)
    h = jnp.dot(y, wr[...], preferred_element_type=jnp.float32)
    o_ref[...] = h * dr[...]

  return pl.pallas_call(
      body, out_shape=jax.ShapeDtypeStruct((N_PAD, D), jnp.float32)
  )(p0, p1, hs, dcol, b, g, be, Wn)


def _tc_post(p0, p1, hs, dcol, b):
  def body(p0r, p1r, hsr, dr, br, o_ref):
    o_ref[...] = (p0r[...] + p1r[...] + hsr[...]) * dr[...] + br[...]

  return pl.pallas_call(
      body, out_shape=jax.ShapeDtypeStruct((N_PAD, D), jnp.float32)
  )(p0, p1, hs, dcol, b)


def kernel(x, edge_index, W1, b1, g1, be1, W2, b2, g2, be2, W3, b3):
  ei = edge_index.astype(jnp.int32)
  src = ei[0].reshape(NW, EPT)
  dst = ei[1].reshape(NW, EPT)

  i = jnp.arange(PADE, dtype=jnp.int32)[None, :]
  w = jnp.arange(NW, dtype=jnp.int32)[:, None]
  src_pad = (i * NW + w) % N
  dst_pad = N + (i * 7 + w) % PADE
  src3 = jnp.concatenate([src, src_pad], axis=1).reshape(NW, NCHUNK, CH)
  dst3 = jnp.concatenate([dst, dst_pad], axis=1).reshape(NW, NCHUNK, CH)

  x_pad = jnp.concatenate(
      [x, jnp.zeros((N_PAD - N, D), dtype=x.dtype)], axis=0)

  sc_deg = _get_sc_deg()
  sc_agg = _get_sc_agg()

  degp = sc_deg(dst3)
  dinv = _tc_dinv(degp)
  dcol = dinv.reshape(N_PAD, 1)

  b1r, g1r, be1r = b1.reshape(1, D), g1.reshape(1, D), be1.reshape(1, D)
  b2r, g2r, be2r = b2.reshape(1, D), g2.reshape(1, D), be2.reshape(1, D)
  b3r = b3.reshape(1, D)

  hs1 = _tc_pre(x_pad, W1, dcol)
  p = sc_agg(hs1, src3, dst3)
  hs2 = _tc_mid(p[0], p[1], hs1, dcol, b1r, g1r, be1r, W2)
  p = sc_agg(hs2, src3, dst3)
  hs3 = _tc_mid(p[0], p[1], hs2, dcol, b2r, g2r, be2r, W3)
  p = sc_agg(hs3, src3, dst3)
  out = _tc_post(p[0], p[1], hs3, dcol, b3r)
  return out[:N]


# speedup vs baseline: 1.0439x; 1.0439x over previous
import functools

import jax
import jax.numpy as jnp
from jax import lax
from jax.experimental import pallas as pl
from jax.experimental.pallas import tpu as pltpu
from jax.experimental.pallas import tpu_sc as plsc

N = 10000
D = 128
E = 320000
NC = 2
NS = 16
NW = NC * NS
EPT = E // NW
CH = 128
NCHUNK = 80
Q = 2
PADE = NCHUNK * CH - EPT
N_PAD = 10240
RPT = N_PAD // NS
EPS = 1e-5


@functools.cache
def _get_sc_deg():
  mesh = plsc.VectorSubcoreMesh(core_axis_name="c", subcore_axis_name="s")

  @functools.partial(
      pl.kernel,
      out_type=jax.ShapeDtypeStruct((NC, N_PAD), jnp.float32),
      mesh=mesh,
      scratch_types=[
          pltpu.VMEM((NCHUNK, CH), jnp.int32),
          pltpu.VMEM((CH,), jnp.float32),
          pltpu.VMEM((RPT,), jnp.float32),
          pltpu.VMEM_SHARED((N_PAD,), jnp.float32),
      ],
  )
  def _sc_deg(dst_hbm, out_hbm, dst_v, ones_v, zbuf_v, acc_sh):
    c = lax.axis_index("c")
    s = lax.axis_index("s")
    wid = c * NS + s

    pltpu.sync_copy(dst_hbm.at[wid], dst_v)

    one16 = jnp.ones((16,), jnp.float32)
    z16 = jnp.zeros((16,), jnp.float32)
    for k in range(CH // 16):
      ones_v[pl.ds(k * 16, 16)] = one16

    def _zero(i, carry):
      zbuf_v[pl.ds(i * 16, 16)] = z16
      return carry

    lax.fori_loop(0, RPT // 16, _zero, 0)
    pltpu.sync_copy(zbuf_v, acc_sh.at[pl.ds(s * RPT, RPT)])
    plsc.subcore_barrier()

    def _chunk(j, carry):
      pltpu.sync_copy(ones_v, acc_sh.at[dst_v.at[j]], add=True)
      return carry

    lax.fori_loop(0, NCHUNK, _chunk, 0)
    plsc.subcore_barrier()

    pltpu.sync_copy(acc_sh.at[pl.ds(s * RPT, RPT)],
                    out_hbm.at[c, pl.ds(s * RPT, RPT)])

  return _sc_deg


@functools.cache
def _get_sc_agg():
  mesh = plsc.VectorSubcoreMesh(core_axis_name="c", subcore_axis_name="s")

  @functools.partial(
      pl.kernel,
      out_type=jax.ShapeDtypeStruct((NC, N_PAD, D), jnp.float32),
      mesh=mesh,
      scratch_types=(
          [pltpu.VMEM((CH,), jnp.int32) for _ in range(Q)]
          + [pltpu.VMEM((CH, D), jnp.float32) for _ in range(Q)]
          + [pltpu.VMEM((NCHUNK, CH), jnp.int32)]
          + [pltpu.VMEM_SHARED((N_PAD, D), jnp.float32)]
          + [pltpu.SemaphoreType.DMA] * (2 * Q)
      ),
  )
  def _sc_agg(hs_hbm, src_hbm, dst_hbm, out_hbm, *refs):
    sidx = refs[0:Q]
    rows = refs[Q:2 * Q]
    dst_v = refs[2 * Q]
    acc_sh = refs[2 * Q + 1]
    sg = refs[2 * Q + 2:2 * Q + 2 + Q]
    si = refs[2 * Q + 2 + Q:2 * Q + 2 + 2 * Q]

    c = lax.axis_index("c")
    s = lax.axis_index("s")
    wid = c * NS + s

    pltpu.sync_copy(dst_hbm.at[wid], dst_v)

    z16 = jnp.zeros((16,), jnp.float32)

    def _zero(i, carry):
      for k in range(D // 16):
        rows[0][i, pl.ds(k * 16, 16)] = z16
      return carry

    lax.fori_loop(0, CH, _zero, 0)
    for m in range(RPT // CH):
      pltpu.sync_copy(rows[0], acc_sh.at[pl.ds(s * RPT + m * CH, CH)])
    if RPT % CH:
      pltpu.sync_copy(
          rows[0].at[pl.ds(0, RPT % CH)],
          acc_sh.at[pl.ds(s * RPT + (RPT // CH) * CH, RPT % CH)])
    plsc.subcore_barrier()

    for q in range(Q - 1):
      pltpu.sync_copy(src_hbm.at[wid, q], sidx[q])
      pltpu.async_copy(hs_hbm.at[sidx[q]], rows[q], sg[q])
    pltpu.async_copy(src_hbm.at[wid, Q - 1], sidx[Q - 1], si[Q - 1])

    def _body(i, carry):
      for k in range(Q):
        t = i * Q + k
        kg = (k + Q - 1) % Q
        tg = jnp.minimum(t + Q - 1, NCHUNK - 1)
        tp = jnp.minimum(t + Q, NCHUNK - 1)
        pltpu.make_async_copy(src_hbm.at[wid, tg], sidx[kg], si[kg]).wait()
        pltpu.async_copy(hs_hbm.at[sidx[kg]], rows[kg], sg[kg])
        pltpu.make_async_copy(hs_hbm.at[sidx[k]], rows[k], sg[k]).wait()
        pltpu.async_copy(src_hbm.at[wid, tp], sidx[k], si[k])
        pltpu.sync_copy(rows[k], acc_sh.at[dst_v.at[t]], add=True)
      return carry

    lax.fori_loop(0, NCHUNK // Q, _body, 0)
    for q in range(Q - 1):
      pltpu.make_async_copy(hs_hbm.at[sidx[q]], rows[q], sg[q]).wait()
    pltpu.make_async_copy(src_hbm.at[wid, 0], sidx[Q - 1], si[Q - 1]).wait()
    plsc.subcore_barrier()

    pltpu.sync_copy(acc_sh.at[pl.ds(s * RPT, RPT)],
                    out_hbm.at[c, pl.ds(s * RPT, RPT)])

  return _sc_agg


def _tc_pre(x, W, partials):
  def body(x_ref, w_ref, p_ref, o_ref, d_ref):
    degc = lax.dot_general(
        p_ref[...], jnp.ones((NC, 1), jnp.float32),
        (((0,), (0,)), ((), ())), preferred_element_type=jnp.float32) + 1.0
    row = lax.broadcasted_iota(jnp.int32, (N_PAD, 1), 0)
    dcol = jnp.where(row < N, lax.rsqrt(degc), 0.0)
    d_ref[...] = dcol
    h = jnp.dot(x_ref[...], w_ref[...], preferred_element_type=jnp.float32)
    o_ref[pl.ds(0, N), :] = h * dcol[:N, :]
    o_ref[pl.ds(N, N_PAD - N), :] = jnp.zeros((N_PAD - N, D), jnp.float32)

  return pl.pallas_call(
      body,
      out_shape=(jax.ShapeDtypeStruct((N_PAD, D), jnp.float32),
                 jax.ShapeDtypeStruct((N_PAD, 1), jnp.float32)),
  )(x, W, partials)


def _tc_mid(p0, p1, hs, dcol, b, g, be, Wn):
  def body(p0r, p1r, hsr, dr, br, gr, ber, wr, o_ref):
    t = (p0r[...] + p1r[...] + hsr[...]) * dr[...] + br[...]
    row = lax.broadcasted_iota(jnp.int32, (N_PAD, 1), 0)
    m = jnp.where(row < N, 1.0, 0.0)
    mean = jnp.sum(t * m, axis=0, keepdims=True) * (1.0 / N)
    ctr = (t - mean) * m
    var = jnp.sum(ctr * ctr, axis=0, keepdims=True) * (1.0 / N)
    y = jnp.maximum(gr[...] * ctr * lax.rsqrt(var + EPS) + ber[...], 0.0)
    h = jnp.dot(y, wr[...], preferred_element_type=jnp.float32)
    o_ref[...] = h * dr[...]

  return pl.pallas_call(
      body, out_shape=jax.ShapeDtypeStruct((N_PAD, D), jnp.float32)
  )(p0, p1, hs, dcol, b, g, be, Wn)


def _tc_post(p0, p1, hs, dcol, b):
  def body(p0r, p1r, hsr, dr, br, o_ref):
    t = (p0r[...] + p1r[...] + hsr[...]) * dr[...] + br[...]
    o_ref[...] = t[:N, :]

  return pl.pallas_call(
      body, out_shape=jax.ShapeDtypeStruct((N, D), jnp.float32)
  )(p0, p1, hs, dcol, b)


def kernel(x, edge_index, W1, b1, g1, be1, W2, b2, g2, be2, W3, b3):
  ei = edge_index.astype(jnp.int32)
  src = ei[0].reshape(NW, EPT)
  dst = ei[1].reshape(NW, EPT)

  i = jnp.arange(PADE, dtype=jnp.int32)[None, :]
  w = jnp.arange(NW, dtype=jnp.int32)[:, None]
  src_pad = (i * NW + w) % N
  dst_pad = N + (i * 7 + w) % PADE
  src3 = jnp.concatenate([src, src_pad], axis=1).reshape(NW, NCHUNK, CH)
  dst3 = jnp.concatenate([dst, dst_pad], axis=1).reshape(NW, NCHUNK, CH)

  sc_deg = _get_sc_deg()
  sc_agg = _get_sc_agg()

  degp = sc_deg(dst3)

  b1r, g1r, be1r = b1.reshape(1, D), g1.reshape(1, D), be1.reshape(1, D)
  b2r, g2r, be2r = b2.reshape(1, D), g2.reshape(1, D), be2.reshape(1, D)
  b3r = b3.reshape(1, D)

  hs1, dcol = _tc_pre(x, W1, degp)
  p = sc_agg(hs1, src3, dst3)
  hs2 = _tc_mid(p[0], p[1], hs1, dcol, b1r, g1r, be1r, W2)
  p = sc_agg(hs2, src3, dst3)
  hs3 = _tc_mid(p[0], p[1], hs2, dcol, b2r, g2r, be2r, W3)
  p = sc_agg(hs3, src3, dst3)
  return _tc_post(p[0], p[1], hs3, dcol, b3r)


# speedup vs baseline: 1.0670x; 1.0221x over previous
import functools

import jax
import jax.numpy as jnp
from jax import lax
from jax.experimental import pallas as pl
from jax.experimental.pallas import tpu as pltpu
from jax.experimental.pallas import tpu_sc as plsc

N = 10000
D = 128
E = 320000
NC = 2
NS = 16
NW = NC * NS
EPT = E // NW
CH = 128
NCHUNK = 80
Q = 2
PADE = NCHUNK * CH - EPT
N_PAD = 10240
RPT = N_PAD // NS
EPS = 1e-5


@functools.cache
def _get_sc_deg():
  mesh = plsc.VectorSubcoreMesh(core_axis_name="c", subcore_axis_name="s")

  @functools.partial(
      pl.kernel,
      out_type=jax.ShapeDtypeStruct((NC, N_PAD), jnp.float32),
      mesh=mesh,
      scratch_types=[
          pltpu.VMEM((NCHUNK, CH), jnp.int32),
          pltpu.VMEM((CH,), jnp.float32),
          pltpu.VMEM((RPT,), jnp.float32),
          pltpu.VMEM_SHARED((N_PAD,), jnp.float32),
          pltpu.SemaphoreType.DMA,
      ],
  )
  def _sc_deg(dst_hbm, out_hbm, dst_v, ones_v, zbuf_v, acc_sh, semd):
    c = lax.axis_index("c")
    s = lax.axis_index("s")
    wid = c * NS + s

    pltpu.sync_copy(dst_hbm.at[wid], dst_v)

    one16 = jnp.ones((16,), jnp.float32)
    z16 = jnp.zeros((16,), jnp.float32)
    for k in range(CH // 16):
      ones_v[pl.ds(k * 16, 16)] = one16

    def _zero(i, carry):
      zbuf_v[pl.ds(i * 16, 16)] = z16
      return carry

    lax.fori_loop(0, RPT // 16, _zero, 0)
    pltpu.sync_copy(zbuf_v, acc_sh.at[pl.ds(s * RPT, RPT)])
    plsc.subcore_barrier()

    WD = 8

    def _fire(j, carry):
      pltpu.async_copy(ones_v, acc_sh.at[dst_v.at[j]], semd, add=True)
      return carry

    def _fire_drain(j, carry):
      pltpu.make_async_copy(ones_v, acc_sh.at[dst_v.at[0]], semd).wait()
      pltpu.async_copy(ones_v, acc_sh.at[dst_v.at[j]], semd, add=True)
      return carry

    def _drain(j, carry):
      pltpu.make_async_copy(ones_v, acc_sh.at[dst_v.at[0]], semd).wait()
      return carry

    lax.fori_loop(0, WD, _fire, 0)
    lax.fori_loop(WD, NCHUNK, _fire_drain, 0)
    lax.fori_loop(0, WD, _drain, 0)
    plsc.subcore_barrier()

    pltpu.sync_copy(acc_sh.at[pl.ds(s * RPT, RPT)],
                    out_hbm.at[c, pl.ds(s * RPT, RPT)])

  return _sc_deg


@functools.cache
def _get_sc_agg():
  mesh = plsc.VectorSubcoreMesh(core_axis_name="c", subcore_axis_name="s")

  @functools.partial(
      pl.kernel,
      out_type=jax.ShapeDtypeStruct((NC, N_PAD, D), jnp.float32),
      mesh=mesh,
      scratch_types=(
          [pltpu.VMEM((CH,), jnp.int32) for _ in range(Q)]
          + [pltpu.VMEM((CH, D), jnp.float32) for _ in range(Q)]
          + [pltpu.VMEM((NCHUNK, CH), jnp.int32)]
          + [pltpu.VMEM_SHARED((N_PAD, D), jnp.float32)]
          + [pltpu.SemaphoreType.DMA] * (2 * Q)
      ),
  )
  def _sc_agg(hs_hbm, src_hbm, dst_hbm, out_hbm, *refs):
    sidx = refs[0:Q]
    rows = refs[Q:2 * Q]
    dst_v = refs[2 * Q]
    acc_sh = refs[2 * Q + 1]
    sg = refs[2 * Q + 2:2 * Q + 2 + Q]
    si = refs[2 * Q + 2 + Q:2 * Q + 2 + 2 * Q]

    c = lax.axis_index("c")
    s = lax.axis_index("s")
    wid = c * NS + s

    pltpu.sync_copy(dst_hbm.at[wid], dst_v)

    for q in range(Q - 1):
      pltpu.sync_copy(src_hbm.at[wid, q], sidx[q])
      pltpu.async_copy(hs_hbm.at[sidx[q]], rows[q], sg[q])
    pltpu.async_copy(src_hbm.at[wid, Q - 1], sidx[Q - 1], si[Q - 1])

    z16 = jnp.zeros((16,), jnp.float32)
    zrow = rows[Q - 1]

    def _zero(i, carry):
      for k in range(D // 16):
        zrow[i, pl.ds(k * 16, 16)] = z16
      return carry

    lax.fori_loop(0, CH, _zero, 0)
    for m in range(RPT // CH):
      pltpu.sync_copy(zrow, acc_sh.at[pl.ds(s * RPT + m * CH, CH)])
    if RPT % CH:
      pltpu.sync_copy(
          zrow.at[pl.ds(0, RPT % CH)],
          acc_sh.at[pl.ds(s * RPT + (RPT // CH) * CH, RPT % CH)])
    plsc.subcore_barrier()

    def _body(i, carry):
      for k in range(Q):
        t = i * Q + k
        kg = (k + Q - 1) % Q
        tg = jnp.minimum(t + Q - 1, NCHUNK - 1)
        tp = jnp.minimum(t + Q, NCHUNK - 1)
        pltpu.make_async_copy(src_hbm.at[wid, tg], sidx[kg], si[kg]).wait()
        pltpu.async_copy(hs_hbm.at[sidx[kg]], rows[kg], sg[kg])
        pltpu.make_async_copy(hs_hbm.at[sidx[k]], rows[k], sg[k]).wait()
        pltpu.async_copy(src_hbm.at[wid, tp], sidx[k], si[k])
        pltpu.sync_copy(rows[k], acc_sh.at[dst_v.at[t]], add=True)
      return carry

    lax.fori_loop(0, NCHUNK // Q, _body, 0)
    for q in range(Q - 1):
      pltpu.make_async_copy(hs_hbm.at[sidx[q]], rows[q], sg[q]).wait()
    pltpu.make_async_copy(src_hbm.at[wid, 0], sidx[Q - 1], si[Q - 1]).wait()
    plsc.subcore_barrier()

    pltpu.sync_copy(acc_sh.at[pl.ds(s * RPT, RPT)],
                    out_hbm.at[c, pl.ds(s * RPT, RPT)])

  return _sc_agg


def _tc_pre(x, W, partials):
  def body(x_ref, w_ref, p_ref, o_ref, d_ref):
    degc = lax.dot_general(
        p_ref[...], jnp.ones((NC, 1), jnp.float32),
        (((0,), (0,)), ((), ())), preferred_element_type=jnp.float32) + 1.0
    row = lax.broadcasted_iota(jnp.int32, (N_PAD, 1), 0)
    dcol = jnp.where(row < N, lax.rsqrt(degc), 0.0)
    d_ref[...] = dcol
    h = jnp.dot(x_ref[...], w_ref[...], preferred_element_type=jnp.float32)
    o_ref[pl.ds(0, N), :] = h * dcol[:N, :]
    o_ref[pl.ds(N, N_PAD - N), :] = jnp.zeros((N_PAD - N, D), jnp.float32)

  return pl.pallas_call(
      body,
      out_shape=(jax.ShapeDtypeStruct((N_PAD, D), jnp.float32),
                 jax.ShapeDtypeStruct((N_PAD, 1), jnp.float32)),
  )(x, W, partials)


def _tc_mid(p0, p1, hs, dcol, b, g, be, Wn):
  def body(p0r, p1r, hsr, dr, br, gr, ber, wr, o_ref):
    t = (p0r[...] + p1r[...] + hsr[...]) * dr[...] + br[...]
    row = lax.broadcasted_iota(jnp.int32, (N_PAD, 1), 0)
    m = jnp.where(row < N, 1.0, 0.0)
    mean = jnp.sum(t * m, axis=0, keepdims=True) * (1.0 / N)
    ctr = (t - mean) * m
    var = jnp.sum(ctr * ctr, axis=0, keepdims=True) * (1.0 / N)
    y = jnp.maximum(gr[...] * ctr * lax.rsqrt(var + EPS) + ber[...], 0.0)
    h = jnp.dot(y, wr[...], preferred_element_type=jnp.float32)
    o_ref[...] = h * dr[...]

  return pl.pallas_call(
      body, out_shape=jax.ShapeDtypeStruct((N_PAD, D), jnp.float32)
  )(p0, p1, hs, dcol, b, g, be, Wn)


def _tc_post(p0, p1, hs, dcol, b):
  def body(p0r, p1r, hsr, dr, br, o_ref):
    t = (p0r[...] + p1r[...] + hsr[...]) * dr[...] + br[...]
    o_ref[...] = t[:N, :]

  return pl.pallas_call(
      body, out_shape=jax.ShapeDtypeStruct((N, D), jnp.float32)
  )(p0, p1, hs, dcol, b)


def kernel(x, edge_index, W1, b1, g1, be1, W2, b2, g2, be2, W3, b3):
  ei = edge_index.astype(jnp.int32)
  src = ei[0].reshape(NW, EPT)
  dst = ei[1].reshape(NW, EPT)

  i = jnp.arange(PADE, dtype=jnp.int32)[None, :]
  w = jnp.arange(NW, dtype=jnp.int32)[:, None]
  src_pad = (i * NW + w) % N
  dst_pad = N + (i * 7 + w) % PADE
  src3 = jnp.concatenate([src, src_pad], axis=1).reshape(NW, NCHUNK, CH)
  dst3 = jnp.concatenate([dst, dst_pad], axis=1).reshape(NW, NCHUNK, CH)

  sc_deg = _get_sc_deg()
  sc_agg = _get_sc_agg()

  degp = sc_deg(dst3)

  b1r, g1r, be1r = b1.reshape(1, D), g1.reshape(1, D), be1.reshape(1, D)
  b2r, g2r, be2r = b2.reshape(1, D), g2.reshape(1, D), be2.reshape(1, D)
  b3r = b3.reshape(1, D)

  hs1, dcol = _tc_pre(x, W1, degp)
  p = sc_agg(hs1, src3, dst3)
  hs2 = _tc_mid(p[0], p[1], hs1, dcol, b1r, g1r, be1r, W2)
  p = sc_agg(hs2, src3, dst3)
  hs3 = _tc_mid(p[0], p[1], hs2, dcol, b2r, g2r, be2r, W3)
  p = sc_agg(hs3, src3, dst3)
  return _tc_post(p[0], p[1], hs3, dcol, b3r)
